# baseline (speedup 1.0000x reference)
import functools

import jax
import jax.numpy as jnp
from jax import lax
from jax.experimental import pallas as pl
from jax.experimental.pallas import tpu as pltpu
from jax.experimental.pallas import tpu_sc as plsc

_FACTORS = 64
_BATCH = 16384
_NC = 2
_NS = 16
_NW = _NC * _NS
_BPW = _BATCH // _NW
_CHUNK = 128
_NCHUNK = _BPW // _CHUNK
_LANES = 16


def _sc_kernel(users_hbm, pos_hbm, neg_hbm, uemb_hbm, iemb_hbm,
               ubias_hbm, ibias_hbm, pos_out, neg_out,
               uidx, pidx, nidx, urows, irows, jrows,
               bu, bi, bj, pos_v, neg_v, sem):
    c = lax.axis_index("c")
    s = lax.axis_index("s")
    wid = s * _NC + c
    base = wid * _BPW

    pltpu.sync_copy(users_hbm.at[pl.ds(base, _BPW)], uidx)
    pltpu.sync_copy(pos_hbm.at[pl.ds(base, _BPW)], pidx)
    pltpu.sync_copy(neg_hbm.at[pl.ds(base, _BPW)], nidx)

    copies = []
    for k in range(_NCHUNK):
        rs = pl.ds(k * _CHUNK, _CHUNK)
        copies.append(pltpu.async_copy(uemb_hbm.at[uidx.at[rs]], urows.at[rs], sem))
        copies.append(pltpu.async_copy(iemb_hbm.at[pidx.at[rs]], irows.at[rs], sem))
        copies.append(pltpu.async_copy(iemb_hbm.at[nidx.at[rs]], jrows.at[rs], sem))
        copies.append(pltpu.async_copy(ubias_hbm.at[uidx.at[rs]], bu.at[rs], sem))
        copies.append(pltpu.async_copy(ibias_hbm.at[pidx.at[rs]], bi.at[rs], sem))
        copies.append(pltpu.async_copy(ibias_hbm.at[nidx.at[rs]], bj.at[rs], sem))
    for cp in copies:
        cp.wait()

    iota = lax.iota(jnp.int32, _LANES)

    def chunk_body(k, carry):
        rows = k * _LANES + iota
        acc_p = jnp.zeros((_LANES,), jnp.float32)
        acc_n = jnp.zeros((_LANES,), jnp.float32)
        for f in range(_FACTORS):
            cols = jnp.full((_LANES,), f, jnp.int32)
            uv = plsc.load_gather(urows, [rows, cols])
            iv = plsc.load_gather(irows, [rows, cols])
            jv = plsc.load_gather(jrows, [rows, cols])
            acc_p = acc_p + uv * iv
            acc_n = acc_n + uv * jv
        sl = pl.ds(k * _LANES, _LANES)
        buv = bu[sl]
        pos_v[sl] = acc_p + buv + bi[sl]
        neg_v[sl] = acc_n + buv + bj[sl]
        return carry

    lax.fori_loop(0, _BPW // _LANES, chunk_body, 0)

    pltpu.sync_copy(pos_v, pos_out.at[pl.ds(base, _BPW)])
    pltpu.sync_copy(neg_v, neg_out.at[pl.ds(base, _BPW)])


@jax.jit
def _run(users, pos_items, neg_items, user_emb, item_emb, user_bias, item_bias):
    mesh = plsc.VectorSubcoreMesh(core_axis_name="c", subcore_axis_name="s")
    kern = functools.partial(
        pl.kernel,
        mesh=mesh,
        compiler_params=pltpu.CompilerParams(needs_layout_passes=False,
                                             use_tc_tiling_on_sc=False),
        out_type=(
            jax.ShapeDtypeStruct((_BATCH,), jnp.float32),
            jax.ShapeDtypeStruct((_BATCH,), jnp.float32),
        ),
        scratch_types=[
            pltpu.VMEM((_BPW,), jnp.int32),
            pltpu.VMEM((_BPW,), jnp.int32),
            pltpu.VMEM((_BPW,), jnp.int32),
            pltpu.VMEM((_BPW, _FACTORS), jnp.float32),
            pltpu.VMEM((_BPW, _FACTORS), jnp.float32),
            pltpu.VMEM((_BPW, _FACTORS), jnp.float32),
            pltpu.VMEM((_BPW,), jnp.float32),
            pltpu.VMEM((_BPW,), jnp.float32),
            pltpu.VMEM((_BPW,), jnp.float32),
            pltpu.VMEM((_BPW,), jnp.float32),
            pltpu.VMEM((_BPW,), jnp.float32),
            pltpu.SemaphoreType.DMA,
        ],
    )(_sc_kernel)
    return kern(users, pos_items, neg_items, user_emb, item_emb,
                user_bias, item_bias)


def kernel(users, pos_items, neg_items, user_emb, item_emb, user_bias, item_bias):
    users = users.astype(jnp.int32)
    pos_items = pos_items.astype(jnp.int32)
    neg_items = neg_items.astype(jnp.int32)
    ub = user_bias.reshape((-1,))
    ib = item_bias.reshape((-1,))
    return _run(users, pos_items, neg_items, user_emb, item_emb, ub, ib)
